# baseline (speedup 1.0000x reference)
import functools

import jax
import jax.numpy as jnp
import numpy as np
from jax.experimental import pallas as pl
from jax.experimental.pallas import tpu as pltpu

NUM_NODES = 1000000
K = 5
SUB = 8
LANES = 2048
TILE = SUB * LANES
NCHUNK = (NUM_NODES + TILE - 1) // TILE
NPAD = NCHUNK * TILE

_U32 = jnp.uint32
_MIN32 = -0x80000000


def _rotl(x, r):
    return (x << _U32(r)) | (x >> _U32(32 - r))


def _threefry(x0, x1):
    ks = (0, 42, (0 ^ 42 ^ 0x1BD11BDA) & 0xFFFFFFFF)
    rot1 = (13, 15, 26, 6)
    rot2 = (17, 29, 16, 24)
    for i in range(5):
        for r in (rot1 if i % 2 == 0 else rot2):
            x0 = x0 + x1
            x1 = _rotl(x1, r)
            x1 = x1 ^ x0
        kx = ks[(i + 1) % 3]
        ky = (ks[(i + 2) % 3] + i + 1) & 0xFFFFFFFF
        if kx:
            x0 = x0 + _U32(kx)
        x1 = x1 + _U32(ky)
    return x0, x1


def _sampler_kernel(q_ref, off_ref, out_ref):
    row = pl.program_id(0) + off_ref[0]
    m = row * 15625
    base_lo = (m << 6).astype(jnp.int32)
    base_hi = m >> 26

    iota_s = jax.lax.broadcasted_iota(jnp.int32, (SUB, LANES), 0)
    iota_l = jax.lax.broadcasted_iota(jnp.int32, (SUB, LANES), 1)
    iota2d = iota_s * LANES + iota_l
    iota2d_u = iota2d.astype(_U32)

    kmin0 = jnp.full((SUB, LANES), 0x7FFFFFFF, jnp.int32)

    def chunk(j, kmin):
        S = base_lo + j * TILE
        wrapped = (S ^ _MIN32) < (base_lo ^ _MIN32)
        hi0 = base_hi + wrapped.astype(jnp.int32)
        hi1 = hi0 + 1
        thresh_raw = -S
        valid = jnp.logical_and(thresh_raw > 0, thresh_raw <= TILE)
        thresh = jnp.where(valid, thresh_raw, jnp.int32(TILE))
        x1 = (iota2d_u + (S + 42).astype(_U32))
        lane_carry = iota2d >= thresh
        x0 = jnp.where(lane_carry, hi1, hi0).astype(_U32)
        b0, b1 = _threefry(x0, x1)
        bits = b0 ^ b1
        fb = (bits >> _U32(9)) | _U32(0x3F800000)
        u = jax.lax.bitcast_convert_type(fb, jnp.float32) - jnp.float32(1.0)
        q = q_ref[pl.ds(j * SUB, SUB), :]
        t = jnp.log2(u) * q
        tb = jax.lax.bitcast_convert_type(t, jnp.int32)
        key = (tb & jnp.int32(-128)) | j
        return jnp.minimum(kmin, key)

    kmin = jax.lax.fori_loop(0, NCHUNK, chunk, kmin0)
    kbest = jnp.min(kmin)
    jwin = kbest & 127
    cand = jnp.where(kmin == kbest, jwin * TILE + iota2d, jnp.int32(2**31 - 1))
    winner = jnp.min(cand)
    out_ref[...] = jnp.full((1, 1, 128), winner, jnp.int32)


def _run_sampler(q2d, row_off, num_rows):
    return pl.pallas_call(
        _sampler_kernel,
        grid=(num_rows,),
        in_specs=[
            pl.BlockSpec((NCHUNK * SUB, LANES), lambda i: (0, 0)),
            pl.BlockSpec(memory_space=pltpu.SMEM),
        ],
        out_specs=pl.BlockSpec((1, 1, 128), lambda i: (i, 0, 0)),
        out_shape=jax.ShapeDtypeStruct((num_rows, 1, 128), jnp.int32),
    )(q2d, row_off)


def kernel(batch, table, probs):
    B = batch.shape[0]
    D = table.shape[1]
    q = 1.0 / probs
    qpad = jnp.full((NPAD,), 1e30, jnp.float32).at[:NUM_NODES].set(q)
    q2d = qpad.reshape(NCHUNK * SUB, LANES)

    devs = jax.devices()
    rows = B * K
    if len(devs) >= 2 and rows % 2 == 0:
        mesh = jax.sharding.Mesh(np.array(devs[:2]), ("x",))
        P = jax.sharding.PartitionSpec

        def _shard_fn(q2d_):
            off = (jax.lax.axis_index("x") * (rows // 2)).astype(jnp.int32)
            return _run_sampler(q2d_, off.reshape(1), rows // 2)

        samp = jax.shard_map(
            _shard_fn, mesh=mesh, in_specs=P(None, None),
            out_specs=P("x", None, None), check_vma=False,
        )(q2d)
    else:
        samp = _run_sampler(q2d, jnp.zeros((1,), jnp.int32), rows)
    neg_idx = samp[:, 0, 0]

    target_embeddings = jnp.take(table, batch[:, 0], axis=0)
    context_embeddings = jnp.take(table, batch[:, 1], axis=0)
    pos_loss = -jax.nn.log_sigmoid(jnp.sum(target_embeddings * context_embeddings, axis=1))
    neg_sample_embeddings = jnp.take(table, neg_idx, axis=0).reshape(B, K, D)
    scores = jnp.squeeze(jnp.matmul(neg_sample_embeddings, target_embeddings[:, :, None]), axis=-1)
    neg_loss = jnp.sum(-jax.nn.log_sigmoid(-scores), axis=1)
    loss = jnp.mean(pos_loss + neg_loss)
    return loss


# speedup vs baseline: 1.0499x; 1.0499x over previous
import functools

import jax
import jax.numpy as jnp
import numpy as np
from jax.experimental import pallas as pl
from jax.experimental.pallas import tpu as pltpu

NUM_NODES = 1000000
K = 5
SUB = 8
LANES = 1024
TILE = SUB * LANES
UNROLL = 2
NCHUNK = -(-NUM_NODES // (TILE * UNROLL)) * UNROLL
NPAD = NCHUNK * TILE

_U32 = jnp.uint32
_MIN32 = -0x80000000


def _rotl(x, r):
    return (x << _U32(r)) | (x >> _U32(32 - r))


def _threefry(x0, x1):
    ks = (0, 42, (0 ^ 42 ^ 0x1BD11BDA) & 0xFFFFFFFF)
    rot1 = (13, 15, 26, 6)
    rot2 = (17, 29, 16, 24)
    for i in range(5):
        for r in (rot1 if i % 2 == 0 else rot2):
            x0 = x0 + x1
            x1 = _rotl(x1, r)
            x1 = x1 ^ x0
        kx = ks[(i + 1) % 3]
        ky = (ks[(i + 2) % 3] + i + 1) & 0xFFFFFFFF
        if kx:
            x0 = x0 + _U32(kx)
        x1 = x1 + _U32(ky)
    return x0, x1


def _sampler_kernel(q_ref, off_ref, out_ref):
    row = pl.program_id(0) + off_ref[0]
    m = row * 15625
    base_lo = (m << 6).astype(jnp.int32)
    base_hi = m >> 26

    iota_s = jax.lax.broadcasted_iota(jnp.int32, (SUB, LANES), 0)
    iota_l = jax.lax.broadcasted_iota(jnp.int32, (SUB, LANES), 1)
    iota2d = iota_s * LANES + iota_l
    iota2d_u = iota2d.astype(_U32)

    kmin0 = jnp.full((SUB, LANES), 0x7FFFFFFF, jnp.int32)

    def eval_chunk(j):
        S = base_lo + j * TILE
        wrapped = (S ^ _MIN32) < (base_lo ^ _MIN32)
        hi0 = base_hi + wrapped.astype(jnp.int32)
        hi1 = hi0 + 1
        thresh_raw = -S
        valid = jnp.logical_and(thresh_raw > 0, thresh_raw <= TILE)
        thresh = jnp.where(valid, thresh_raw, jnp.int32(TILE))
        x1 = (iota2d_u + (S + 42).astype(_U32))
        lane_carry = iota2d >= thresh
        x0 = jnp.where(lane_carry, hi1, hi0).astype(_U32)
        b0, b1 = _threefry(x0, x1)
        bits = b0 ^ b1
        fb = (bits >> _U32(9)) | _U32(0x3F800000)
        u = jax.lax.bitcast_convert_type(fb, jnp.float32) - jnp.float32(1.0)
        q = q_ref[pl.ds(j * SUB, SUB), :]
        t = jnp.log2(u) * q
        tb = jax.lax.bitcast_convert_type(t, jnp.int32)
        return (tb & jnp.int32(-128)) | j

    def step(i, kmin):
        j = i * UNROLL
        k = jnp.minimum(eval_chunk(j), eval_chunk(j + 1))
        return jnp.minimum(kmin, k)

    kmin = jax.lax.fori_loop(0, NCHUNK // UNROLL, step, kmin0)
    kbest = jnp.min(kmin)
    jwin = kbest & 127
    cand = jnp.where(kmin == kbest, jwin * TILE + iota2d, jnp.int32(2**31 - 1))
    winner = jnp.min(cand)
    out_ref[...] = jnp.full((1, 1, 128), winner, jnp.int32)


def _run_sampler(q2d, row_off, num_rows):
    return pl.pallas_call(
        _sampler_kernel,
        grid=(num_rows,),
        in_specs=[
            pl.BlockSpec((NCHUNK * SUB, LANES), lambda i: (0, 0)),
            pl.BlockSpec(memory_space=pltpu.SMEM),
        ],
        out_specs=pl.BlockSpec((1, 1, 128), lambda i: (i, 0, 0)),
        out_shape=jax.ShapeDtypeStruct((num_rows, 1, 128), jnp.int32),
    )(q2d, row_off)


def kernel(batch, table, probs):
    B = batch.shape[0]
    D = table.shape[1]
    q = 1.0 / probs
    qpad = jnp.full((NPAD,), 1e30, jnp.float32).at[:NUM_NODES].set(q)
    q2d = qpad.reshape(NCHUNK * SUB, LANES)

    devs = jax.devices()
    rows = B * K
    if len(devs) >= 2 and rows % 2 == 0:
        mesh = jax.sharding.Mesh(np.array(devs[:2]), ("x",))
        P = jax.sharding.PartitionSpec

        def _shard_fn(q2d_):
            off = (jax.lax.axis_index("x") * (rows // 2)).astype(jnp.int32)
            return _run_sampler(q2d_, off.reshape(1), rows // 2)

        samp = jax.shard_map(
            _shard_fn, mesh=mesh, in_specs=P(None, None),
            out_specs=P("x", None, None), check_vma=False,
        )(q2d)
    else:
        samp = _run_sampler(q2d, jnp.zeros((1,), jnp.int32), rows)
    neg_idx = samp[:, 0, 0]

    target_embeddings = jnp.take(table, batch[:, 0], axis=0)
    context_embeddings = jnp.take(table, batch[:, 1], axis=0)
    pos_loss = -jax.nn.log_sigmoid(jnp.sum(target_embeddings * context_embeddings, axis=1))
    neg_sample_embeddings = jnp.take(table, neg_idx, axis=0).reshape(B, K, D)
    scores = jnp.squeeze(jnp.matmul(neg_sample_embeddings, target_embeddings[:, :, None]), axis=-1)
    neg_loss = jnp.sum(-jax.nn.log_sigmoid(-scores), axis=1)
    loss = jnp.mean(pos_loss + neg_loss)
    return loss
